# baseline (speedup 1.0000x reference)
import jax
import jax.numpy as jnp
from jax import lax
from jax.experimental import pallas as pl
from jax.experimental.pallas import tpu as pltpu
from jax.experimental.pallas import tpu_sc as plsc

TOKENS = 16384
EXPERTS = 64
K = 8
LANES = 16

NUM_CORES = 2
NUM_SUBCORES = 16
NUM_WORKERS = NUM_CORES * NUM_SUBCORES
TPT = TOKENS // NUM_WORKERS
GROUPS = TPT // LANES

_SORT8 = (
    (0, 1), (2, 3), (4, 5), (6, 7),
    (0, 2), (1, 3), (4, 6), (5, 7),
    (1, 2), (5, 6),
    (0, 4), (1, 5), (2, 6), (3, 7),
    (2, 4), (3, 5),
    (1, 2), (3, 4), (5, 6),
)
_BITONIC8 = (
    (0, 4), (1, 5), (2, 6), (3, 7),
    (0, 2), (1, 3), (4, 6), (5, 7),
    (0, 1), (2, 3), (4, 5), (6, 7),
)


def _ce(keys, ids, i, j):
    c = keys[j] > keys[i]
    ki = jnp.maximum(keys[i], keys[j])
    kj = jnp.minimum(keys[i], keys[j])
    ii, ij = jnp.where(c, ids[j], ids[i]), jnp.where(c, ids[i], ids[j])
    keys[i], keys[j] = ki, kj
    ids[i], ids[j] = ii, ij


def _sort8(keys, ids):
    for i, j in _SORT8:
        _ce(keys, ids, i, j)


def _body(xt_hbm, w_hbm, i_hbm, x_vmem, w_vmem, i_vmem):
    wid = lax.axis_index("s") * NUM_CORES + lax.axis_index("c")
    base = wid * TPT
    pltpu.sync_copy(xt_hbm.at[:, pl.ds(base, TPT)], x_vmem)

    @plsc.parallel_loop(0, GROUPS, step=1, unroll=4)
    def group_body(g):
        col = g * LANES

        def load8(g8):
            keys = [x_vmem[8 * g8 + e, pl.ds(col, LANES)] for e in range(8)]
            ids = [jnp.full((LANES,), 8 * g8 + e, jnp.int32) for e in range(8)]
            _sort8(keys, ids)
            return keys, ids

        rk, ri = load8(0)
        for g8 in range(1, EXPERTS // 8):
            sk, si = load8(g8)
            tk, ti = [], []
            for i in range(8):
                c = sk[7 - i] > rk[i]
                tk.append(jnp.where(c, sk[7 - i], rk[i]))
                ti.append(jnp.where(c, si[7 - i], ri[i]))
            for i, j in _BITONIC8:
                _ce(tk, ti, i, j)
            rk, ri = tk, ti

        z = [jnp.ones((LANES,), jnp.float32)]
        z += [jnp.exp(rk[j] - rk[0]) for j in range(1, 8)]
        s = (z[0] + z[1]) + (z[2] + z[3]) + ((z[4] + z[5]) + (z[6] + z[7]))
        r = 1.0 / s
        for j in range(8):
            w_vmem[j, pl.ds(col, LANES)] = z[j] * r
            i_vmem[j, pl.ds(col, LANES)] = ri[j]

    pltpu.sync_copy(w_vmem, w_hbm.at[:, pl.ds(base, TPT)])
    pltpu.sync_copy(i_vmem, i_hbm.at[:, pl.ds(base, TPT)])


@jax.jit
def kernel(router_logits):
    mesh = plsc.VectorSubcoreMesh(core_axis_name="c", subcore_axis_name="s")
    wt, it = pl.kernel(
        _body,
        out_type=[
            jax.ShapeDtypeStruct((K, TOKENS), jnp.float32),
            jax.ShapeDtypeStruct((K, TOKENS), jnp.int32),
        ],
        mesh=mesh,
        compiler_params=pltpu.CompilerParams(
            needs_layout_passes=False,
            disable_bounds_checks=True,
            disable_semaphore_checks=True,
        ),
        scratch_types=[
            pltpu.VMEM((EXPERTS, TPT), jnp.float32),
            pltpu.VMEM((K, TPT), jnp.float32),
            pltpu.VMEM((K, TPT), jnp.int32),
        ],
    )(router_logits.T)
    return wt.T, it.T


# speedup vs baseline: 1.4621x; 1.4621x over previous
import jax
import jax.numpy as jnp
from jax import lax
from jax.experimental import pallas as pl
from jax.experimental.pallas import tpu as pltpu
from jax.experimental.pallas import tpu_sc as plsc

TOKENS = 16384
EXPERTS = 64
K = 8
LANES = 16

NUM_CORES = 2
NUM_SUBCORES = 16
NUM_WORKERS = NUM_CORES * NUM_SUBCORES
TPT = TOKENS // NUM_WORKERS
GROUPS = TPT // LANES

_SORT8 = (
    (0, 1), (2, 3), (4, 5), (6, 7),
    (0, 2), (1, 3), (4, 6), (5, 7),
    (1, 2), (5, 6),
    (0, 4), (1, 5), (2, 6), (3, 7),
    (2, 4), (3, 5),
    (1, 2), (3, 4), (5, 6),
)
_BITONIC8 = (
    (0, 4), (1, 5), (2, 6), (3, 7),
    (0, 2), (1, 3), (4, 6), (5, 7),
    (0, 1), (2, 3), (4, 5), (6, 7),
)


def _ce(keys, ids, i, j):
    c = keys[j] > keys[i]
    ki = jnp.maximum(keys[i], keys[j])
    kj = jnp.minimum(keys[i], keys[j])
    ii, ij = jnp.where(c, ids[j], ids[i]), jnp.where(c, ids[i], ids[j])
    keys[i], keys[j] = ki, kj
    ids[i], ids[j] = ii, ij


def _sort8(keys, ids):
    for i, j in _SORT8:
        _ce(keys, ids, i, j)


def _body(xt_hbm, w_hbm, i_hbm, x_vmem, w_vmem, i_vmem):
    wid = lax.axis_index("s") * NUM_CORES + lax.axis_index("c")
    base = wid * TPT
    pltpu.sync_copy(xt_hbm.at[:, pl.ds(base, TPT)], x_vmem)

    @plsc.parallel_loop(0, GROUPS, step=1, unroll=2)
    def group_body(g):
        col = g * LANES

        def load8(g8):
            keys = [x_vmem[8 * g8 + e, pl.ds(col, LANES)] for e in range(8)]
            ids = [jnp.full((LANES,), 8 * g8 + e, jnp.int32) for e in range(8)]
            _sort8(keys, ids)
            return keys, ids

        rk, ri = load8(0)
        for g8 in range(1, EXPERTS // 8):
            sk, si = load8(g8)
            tk, ti = [], []
            for i in range(8):
                c = sk[7 - i] > rk[i]
                tk.append(jnp.where(c, sk[7 - i], rk[i]))
                ti.append(jnp.where(c, si[7 - i], ri[i]))
            for i, j in _BITONIC8:
                _ce(tk, ti, i, j)
            rk, ri = tk, ti

        z = [jnp.ones((LANES,), jnp.float32)]
        z += [jnp.exp(rk[j] - rk[0]) for j in range(1, 8)]
        s = (z[0] + z[1]) + (z[2] + z[3]) + ((z[4] + z[5]) + (z[6] + z[7]))
        r = 1.0 / s
        for j in range(8):
            w_vmem[j, pl.ds(col, LANES)] = z[j] * r
            i_vmem[j, pl.ds(col, LANES)] = ri[j]

    pltpu.sync_copy(w_vmem, w_hbm.at[:, pl.ds(base, TPT)])
    pltpu.sync_copy(i_vmem, i_hbm.at[:, pl.ds(base, TPT)])


@jax.jit
def kernel(router_logits):
    mesh = plsc.VectorSubcoreMesh(core_axis_name="c", subcore_axis_name="s")
    wt, it = pl.kernel(
        _body,
        out_type=[
            jax.ShapeDtypeStruct((K, TOKENS), jnp.float32),
            jax.ShapeDtypeStruct((K, TOKENS), jnp.int32),
        ],
        mesh=mesh,
        compiler_params=pltpu.CompilerParams(
            needs_layout_passes=False,
            disable_bounds_checks=True,
            disable_semaphore_checks=True,
        ),
        scratch_types=[
            pltpu.VMEM((EXPERTS, TPT), jnp.float32),
            pltpu.VMEM((K, TPT), jnp.float32),
            pltpu.VMEM((K, TPT), jnp.int32),
        ],
    )(router_logits.T)
    return wt.T, it.T


# speedup vs baseline: 1.4626x; 1.0004x over previous
import jax
import jax.numpy as jnp
from jax import lax
from jax.experimental import pallas as pl
from jax.experimental.pallas import tpu as pltpu
from jax.experimental.pallas import tpu_sc as plsc

TOKENS = 16384
EXPERTS = 64
K = 8
LANES = 16

NUM_CORES = 2
NUM_SUBCORES = 16
NUM_WORKERS = NUM_CORES * NUM_SUBCORES
TPT = TOKENS // NUM_WORKERS
GROUPS = TPT // LANES

_SORT8 = (
    (0, 1), (2, 3), (4, 5), (6, 7),
    (0, 2), (1, 3), (4, 6), (5, 7),
    (1, 2), (5, 6),
    (0, 4), (1, 5), (2, 6), (3, 7),
    (2, 4), (3, 5),
    (1, 2), (3, 4), (5, 6),
)
_BITONIC8 = (
    (0, 4), (1, 5), (2, 6), (3, 7),
    (0, 2), (1, 3), (4, 6), (5, 7),
    (0, 1), (2, 3), (4, 5), (6, 7),
)


def _ce(keys, ids, i, j):
    c = keys[j] > keys[i]
    ki = jnp.maximum(keys[i], keys[j])
    kj = jnp.minimum(keys[i], keys[j])
    ii, ij = jnp.where(c, ids[j], ids[i]), jnp.where(c, ids[i], ids[j])
    keys[i], keys[j] = ki, kj
    ids[i], ids[j] = ii, ij


def _sort8(keys, ids):
    for i, j in _SORT8:
        _ce(keys, ids, i, j)


def _body(xt_hbm, w_hbm, i_hbm, x_vmem, w_vmem, i_vmem, osem):
    wid = lax.axis_index("s") * NUM_CORES + lax.axis_index("c")
    base = wid * TPT
    pltpu.sync_copy(xt_hbm.at[:, pl.ds(base, TPT)], x_vmem)

    @plsc.parallel_loop(0, GROUPS, step=1, unroll=2)
    def group_body(g):
        col = g * LANES

        def load8(g8):
            keys = [x_vmem[8 * g8 + e, pl.ds(col, LANES)] for e in range(8)]
            ids = [jnp.full((LANES,), 8 * g8 + e, jnp.int32) for e in range(8)]
            _sort8(keys, ids)
            return keys, ids

        rk, ri = load8(0)
        for g8 in range(1, EXPERTS // 8):
            sk, si = load8(g8)
            tk, ti = [], []
            for i in range(8):
                c = sk[7 - i] > rk[i]
                tk.append(jnp.where(c, sk[7 - i], rk[i]))
                ti.append(jnp.where(c, si[7 - i], ri[i]))
            for i, j in _BITONIC8:
                _ce(tk, ti, i, j)
            rk, ri = tk, ti

        z = [jnp.ones((LANES,), jnp.float32)]
        z += [jnp.exp(rk[j] - rk[0]) for j in range(1, 8)]
        s = (z[0] + z[1]) + (z[2] + z[3]) + ((z[4] + z[5]) + (z[6] + z[7]))
        r = 1.0 / s
        for j in range(8):
            w_vmem[j, pl.ds(col, LANES)] = z[j] * r
            i_vmem[j, pl.ds(col, LANES)] = ri[j]

    cpw = pltpu.make_async_copy(w_vmem, w_hbm.at[:, pl.ds(base, TPT)], osem)
    cpi = pltpu.make_async_copy(i_vmem, i_hbm.at[:, pl.ds(base, TPT)], osem)
    cpw.start()
    cpi.start()
    cpw.wait()
    cpi.wait()


@jax.jit
def kernel(router_logits):
    mesh = plsc.VectorSubcoreMesh(core_axis_name="c", subcore_axis_name="s")
    wt, it = pl.kernel(
        _body,
        out_type=[
            jax.ShapeDtypeStruct((K, TOKENS), jnp.float32),
            jax.ShapeDtypeStruct((K, TOKENS), jnp.int32),
        ],
        mesh=mesh,
        compiler_params=pltpu.CompilerParams(
            needs_layout_passes=False,
            disable_bounds_checks=True,
            disable_semaphore_checks=True,
        ),
        scratch_types=[
            pltpu.VMEM((EXPERTS, TPT), jnp.float32),
            pltpu.VMEM((K, TPT), jnp.float32),
            pltpu.VMEM((K, TPT), jnp.int32),
            pltpu.SemaphoreType.DMA,
        ],
    )(router_logits.T)
    return wt.T, it.T
